# baseline (speedup 1.0000x reference)
import functools

import jax
import jax.numpy as jnp
from jax import lax
from jax.experimental import pallas as pl
from jax.experimental.pallas import tpu as pltpu
from jax.experimental.pallas import tpu_sc as plsc

N_NODES = 10000
N_EDGES = 320000
D = 128
L = 16

NC = 2
NS = 16
NW = NC * NS
EPW = 10240
K = 64
CHUNKS = EPW // K
NBUF = 5
LOOK = NBUF - 2
STEPS = CHUNKS // NBUF
RPT = 624
TAIL = N_NODES - RPT * NS
ZROWS = 8


def _sc_scatter(x, src, dst, vals):
    mesh = plsc.VectorSubcoreMesh(core_axis_name="c", subcore_axis_name="s")

    rows_t = [pltpu.VMEM((K, D), jnp.float32) for _ in range(NBUF)]
    srcb_t = [pltpu.VMEM((K,), jnp.int32) for _ in range(NBUF)]
    dstb_t = [pltpu.VMEM((K,), jnp.int32) for _ in range(NBUF)]
    valb_t = [pltpu.VMEM((K,), jnp.float32) for _ in range(NBUF)]
    sems_t = [pltpu.SemaphoreType.DMA for _ in range(3 * NBUF)]

    @functools.partial(
        pl.kernel,
        out_type=jax.ShapeDtypeStruct((NC, N_NODES, D), jnp.float32),
        mesh=mesh,
        scratch_types=(rows_t + srcb_t + dstb_t + valb_t
                       + [pltpu.VMEM((ZROWS, D), jnp.float32),
                          pltpu.VMEM_SHARED((N_NODES, D), jnp.float32)]
                       + sems_t),
    )
    def k(x_hbm, src_hbm, dst_hbm, vals_hbm, out_hbm, *refs):
        rows = refs[0:NBUF]
        srcb = refs[NBUF:2 * NBUF]
        dstb = refs[2 * NBUF:3 * NBUF]
        valb = refs[3 * NBUF:4 * NBUF]
        zero_v = refs[4 * NBUF]
        acc_sh = refs[4 * NBUF + 1]
        rsem = refs[4 * NBUF + 2:4 * NBUF + 2 + NBUF]
        ssem = refs[4 * NBUF + 2 + NBUF:4 * NBUF + 2 + 2 * NBUF]
        msem = refs[4 * NBUF + 2 + 2 * NBUF:4 * NBUF + 2 + 3 * NBUF]

        cid = lax.axis_index("c")
        sid = lax.axis_index("s")
        wid = sid * NC + cid
        base0 = wid * EPW

        def meta_fetch(c, q):
            sl = pl.ds(base0 + c * K, K)
            pltpu.async_copy(src_hbm.at[sl], srcb[q], msem[q])
            pltpu.async_copy(dst_hbm.at[sl], dstb[q], msem[q])
            pltpu.async_copy(vals_hbm.at[sl], valb[q], msem[q])

        def meta_wait(c, q):
            sl = pl.ds(base0 + c * K, K)
            pltpu.make_async_copy(src_hbm.at[sl], srcb[q], msem[q]).wait()
            pltpu.make_async_copy(dst_hbm.at[sl], dstb[q], msem[q]).wait()
            pltpu.make_async_copy(vals_hbm.at[sl], valb[q], msem[q]).wait()

        def gather(q):
            pltpu.async_copy(x_hbm.at[srcb[q]], rows[q], rsem[q])

        def gather_wait(q):
            pltpu.make_async_copy(x_hbm.at[srcb[q]], rows[q],
                                  rsem[q]).wait()

        def scatter(q):
            pltpu.async_copy(rows[q], acc_sh.at[dstb[q]], ssem[q], add=True)

        def scatter_wait(q):
            pltpu.make_async_copy(rows[q], acc_sh.at[dstb[q]],
                                  ssem[q]).wait()

        def scale(q):
            buf = rows[q]
            vbuf = valb[q]

            def body(g, _):
                vv = vbuf[pl.ds(g * L, L)]
                for i in range(L):
                    v = vv[i]
                    e = g * L + i
                    for j in range(D // L):
                        sl = pl.ds(j * L, L)
                        buf[e, sl] = buf[e, sl] * v
                return 0

            lax.fori_loop(0, K // L, body, 0)

        for c in range(LOOK):
            meta_fetch(c, c)

        zvec = jnp.zeros((L,), jnp.float32)
        for j in range(D // L):
            for i in range(ZROWS):
                zero_v[i, pl.ds(j * L, L)] = zvec
        row0 = pl.multiple_of(sid * RPT, 8)

        def zcopy(t, _):
            off = pl.multiple_of(row0 + t * ZROWS, 8)
            pltpu.sync_copy(zero_v, acc_sh.at[pl.ds(off, ZROWS)])
            return 0

        lax.fori_loop(0, RPT // ZROWS, zcopy, 0)

        @pl.when(sid == NS - 1)
        def _zero_tail():
            for t in range(TAIL // ZROWS):
                pltpu.sync_copy(
                    zero_v, acc_sh.at[pl.ds(RPT * NS + t * ZROWS, ZROWS)])

        for c in range(LOOK):
            meta_wait(c, c)
            gather(c)
        plsc.subcore_barrier()

        def step(s, _):
            c0 = s * NBUF
            for q in range(NBUF):
                c = c0 + q
                f = (q + LOOK) % NBUF

                @pl.when(c + LOOK < CHUNKS)
                def _prefetch():
                    meta_fetch(c + LOOK, f)

                gather_wait(q)
                scale(q)

                @pl.when(c >= 1)
                def _drain_prev():
                    scatter_wait((q + NBUF - 1) % NBUF)

                scatter(q)

                @pl.when(c + LOOK < CHUNKS)
                def _launch():
                    meta_wait(c + LOOK, f)
                    gather(f)

            return 0

        lax.fori_loop(0, STEPS, step, 0)
        scatter_wait((CHUNKS - 1) % NBUF)
        plsc.subcore_barrier()

        pltpu.sync_copy(acc_sh.at[pl.ds(row0, RPT)],
                        out_hbm.at[cid, pl.ds(row0, RPT)])

        @pl.when(sid == NS - 1)
        def _write_tail():
            pltpu.sync_copy(acc_sh.at[pl.ds(RPT * NS, TAIL)],
                            out_hbm.at[cid, pl.ds(RPT * NS, TAIL)])

    return k(x, src, dst, vals)


def _tc_combine(partials, W, b):
    R = 1000

    def body(p0_ref, p1_ref, w_ref, b_ref, o_ref):
        s = p0_ref[...] + p1_ref[...]
        y = jnp.dot(s, w_ref[...], preferred_element_type=jnp.float32)
        o_ref[...] = jnp.maximum(y + b_ref[...], 0.0)

    return pl.pallas_call(
        body,
        grid=(N_NODES // R,),
        in_specs=[
            pl.BlockSpec((R, D), lambda i: (i, 0)),
            pl.BlockSpec((R, D), lambda i: (i, 0)),
            pl.BlockSpec((D, D), lambda i: (0, 0)),
            pl.BlockSpec((1, D), lambda i: (0, 0)),
        ],
        out_specs=pl.BlockSpec((R, D), lambda i: (i, 0)),
        out_shape=jax.ShapeDtypeStruct((N_NODES, D), jnp.float32),
    )(partials[0], partials[1], W, b.reshape(1, D))


def _pad_worker_slices(a):
    per = N_EDGES // NW
    return jnp.pad(a.reshape(NW, per), ((0, 0), (0, EPW - per))).reshape(-1)


def kernel(x, edge_index, edge_vals, W, b):
    src = _pad_worker_slices(edge_index[0].astype(jnp.int32))
    dst = _pad_worker_slices(edge_index[1].astype(jnp.int32))
    vals = _pad_worker_slices(edge_vals.astype(jnp.float32))
    partials = _sc_scatter(x, src, dst, vals)
    return _tc_combine(partials, W, b)


# speedup vs baseline: 1.3308x; 1.3308x over previous
import functools

import jax
import jax.numpy as jnp
from jax import lax
from jax.experimental import pallas as pl
from jax.experimental.pallas import tpu as pltpu
from jax.experimental.pallas import tpu_sc as plsc

N_NODES = 10000
N_EDGES = 320000
D = 128
L = 16

NC = 2
NS = 16
NW = NC * NS
EPW = N_EDGES // NW
K = 80
CHUNKS = EPW // K
NBUF = 4
STEPS = (CHUNKS - 1) // NBUF
RPT = 624
TAIL = N_NODES - RPT * NS
ZROWS = 16
MASK_HI = -65536


def _sc_scatter(xbf, src, dst, vals):
    mesh = plsc.VectorSubcoreMesh(core_axis_name="c", subcore_axis_name="s")

    rows_t = [pltpu.VMEM((K, D // 2), jnp.int32) for _ in range(NBUF)]
    st_t = [pltpu.VMEM((K, D), jnp.float32) for _ in range(2)]
    srcb_t = [pltpu.VMEM((K,), jnp.int32) for _ in range(NBUF)]
    dstb_t = [pltpu.VMEM((K,), jnp.int32) for _ in range(NBUF)]
    valb_t = [pltpu.VMEM((K,), jnp.float32) for _ in range(NBUF)]
    sems_t = [pltpu.SemaphoreType.DMA for _ in range(2 * NBUF + 2)]

    @functools.partial(
        pl.kernel,
        out_type=jax.ShapeDtypeStruct((NC, N_NODES, D), jnp.float32),
        mesh=mesh,
        compiler_params=pltpu.CompilerParams(use_tc_tiling_on_sc=False,
                                             needs_layout_passes=False),
        scratch_types=(rows_t + st_t + srcb_t + dstb_t + valb_t
                       + [pltpu.VMEM((ZROWS, D), jnp.float32),
                          pltpu.VMEM_SHARED((N_NODES, D), jnp.float32)]
                       + sems_t),
    )
    def k(x_hbm, src_hbm, dst_hbm, vals_hbm, out_hbm, *refs):
        rows = refs[0:NBUF]
        st32 = refs[NBUF:NBUF + 2]
        srcb = refs[NBUF + 2:2 * NBUF + 2]
        dstb = refs[2 * NBUF + 2:3 * NBUF + 2]
        valb = refs[3 * NBUF + 2:4 * NBUF + 2]
        zero_v = refs[4 * NBUF + 2]
        acc_sh = refs[4 * NBUF + 3]
        rsem = refs[4 * NBUF + 4:5 * NBUF + 4]
        msem = refs[5 * NBUF + 4:6 * NBUF + 4]
        ssem = refs[6 * NBUF + 4:6 * NBUF + 6]

        cid = lax.axis_index("c")
        sid = lax.axis_index("s")
        wid = sid * NC + cid
        base0 = wid * EPW

        iota = lax.iota(jnp.int32, L)
        cols_lo = [iota * 2 + j * 32 for j in range(D // 32)]
        cols_hi = [iota * 2 + (j * 32 + 1) for j in range(D // 32)]

        def meta_fetch(c, q):
            sl = pl.ds(base0 + c * K, K)
            pltpu.async_copy(src_hbm.at[sl], srcb[q], msem[q])
            pltpu.async_copy(dst_hbm.at[sl], dstb[q], msem[q])
            pltpu.async_copy(vals_hbm.at[sl], valb[q], msem[q])

        def meta_wait(c, q):
            sl = pl.ds(base0 + c * K, K)
            pltpu.make_async_copy(src_hbm.at[sl], srcb[q], msem[q]).wait()
            pltpu.make_async_copy(dst_hbm.at[sl], dstb[q], msem[q]).wait()
            pltpu.make_async_copy(vals_hbm.at[sl], valb[q], msem[q]).wait()

        def gather(q):
            pltpu.async_copy(x_hbm.at[srcb[q]], rows[q], rsem[q])

        def gather_wait(q):
            pltpu.make_async_copy(x_hbm.at[srcb[q]], rows[q],
                                  rsem[q]).wait()

        def scatter(q, par):
            pltpu.async_copy(st32[par], acc_sh.at[dstb[q]], ssem[par],
                             add=True)

        def scatter_wait(q, par):
            pltpu.make_async_copy(st32[par], acc_sh.at[dstb[q]],
                                  ssem[par]).wait()

        def scale(q, par):
            buf = rows[q]
            out = st32[par]
            vbuf = valb[q]

            def body(g, _):
                vv = vbuf[pl.ds(g * L, L)]
                for i in range(L):
                    v = vv[i]
                    e = g * L + i
                    e_vec = jnp.full((L,), e, jnp.int32)
                    for j in range(D // 32):
                        p = buf[e, pl.ds(j * L, L)]
                        lo = lax.bitcast_convert_type(
                            p << 16, jnp.float32) * v
                        hi = lax.bitcast_convert_type(
                            p & MASK_HI, jnp.float32) * v
                        plsc.store_scatter(out, [e_vec, cols_lo[j]], lo)
                        plsc.store_scatter(out, [e_vec, cols_hi[j]], hi)
                return 0

            lax.fori_loop(0, K // L, body, 0)

        meta_fetch(0, 0)
        meta_fetch(1, 1)

        zvec = jnp.zeros((L,), jnp.float32)
        for j in range(D // L):
            for i in range(ZROWS):
                zero_v[i, pl.ds(j * L, L)] = zvec
        row0 = pl.multiple_of(sid * RPT, 8)

        def zcopy(t, _):
            off = pl.multiple_of(row0 + t * ZROWS, 8)
            pltpu.sync_copy(zero_v, acc_sh.at[pl.ds(off, ZROWS)])
            return 0

        lax.fori_loop(0, RPT // ZROWS, zcopy, 0)

        @pl.when(sid == NS - 1)
        def _zero_tail():
            pltpu.sync_copy(zero_v, acc_sh.at[pl.ds(RPT * NS, TAIL)])

        meta_wait(0, 0)
        gather(0)
        meta_wait(1, 1)
        gather(1)
        plsc.subcore_barrier()

        def step(s, _):
            c0 = s * NBUF
            for q in range(NBUF):
                c = c0 + q
                f = (q + 2) % NBUF
                par = q % 2

                @pl.when(c + 2 < CHUNKS)
                def _prefetch():
                    meta_fetch(c + 2, f)

                gather_wait(q)
                scale(q, par)

                @pl.when(c >= 1)
                def _drain_prev():
                    scatter_wait((q + NBUF - 1) % NBUF, (par + 1) % 2)

                scatter(q, par)

                @pl.when(c + 2 < CHUNKS)
                def _launch():
                    meta_wait(c + 2, f)
                    gather(f)

            return 0

        lax.fori_loop(0, STEPS, step, 0)

        gather_wait(0)
        scale(0, 0)
        scatter_wait(3, 1)
        scatter(0, 0)
        scatter_wait(0, 0)
        plsc.subcore_barrier()

        pltpu.sync_copy(acc_sh.at[pl.ds(row0, RPT)],
                        out_hbm.at[cid, pl.ds(row0, RPT)])

        @pl.when(sid == NS - 1)
        def _write_tail():
            pltpu.sync_copy(acc_sh.at[pl.ds(RPT * NS, TAIL)],
                            out_hbm.at[cid, pl.ds(RPT * NS, TAIL)])

    return k(xbf, src, dst, vals)


def _tc_combine(partials, W, b):
    R = 1000

    def body(p0_ref, p1_ref, w_ref, b_ref, o_ref):
        s = p0_ref[...] + p1_ref[...]
        y = jnp.dot(s, w_ref[...], preferred_element_type=jnp.float32)
        o_ref[...] = jnp.maximum(y + b_ref[...], 0.0)

    return pl.pallas_call(
        body,
        grid=(N_NODES // R,),
        in_specs=[
            pl.BlockSpec((R, D), lambda i: (i, 0)),
            pl.BlockSpec((R, D), lambda i: (i, 0)),
            pl.BlockSpec((D, D), lambda i: (0, 0)),
            pl.BlockSpec((1, D), lambda i: (0, 0)),
        ],
        out_specs=pl.BlockSpec((R, D), lambda i: (i, 0)),
        out_shape=jax.ShapeDtypeStruct((N_NODES, D), jnp.float32),
    )(partials[0], partials[1], W, b.reshape(1, D))


def kernel(x, edge_index, edge_vals, W, b):
    src = edge_index[0].astype(jnp.int32)
    dst = edge_index[1].astype(jnp.int32)
    xbits = lax.bitcast_convert_type(
        x.astype(jnp.bfloat16).reshape(N_NODES, D // 2, 2), jnp.int32)
    partials = _sc_scatter(xbits, src, dst, edge_vals.astype(jnp.float32))
    return _tc_combine(partials, W, b)


# speedup vs baseline: 2.7271x; 2.0492x over previous
import functools

import jax
import jax.numpy as jnp
from jax import lax
from jax.experimental import pallas as pl
from jax.experimental.pallas import tpu as pltpu
from jax.experimental.pallas import tpu_sc as plsc

N_NODES = 10000
N_EDGES = 320000
D = 128
L = 16

NC = 2
NS = 16
NW = NC * NS
EPW = N_EDGES // NW
K = 80
CHUNKS = EPW // K
NBUF = 4
STEPS = (CHUNKS - 1) // NBUF
RPT = 624
TAIL = N_NODES - RPT * NS
ZROWS = 16


def _sc_scatter(x, src, dst, vals):
    mesh = plsc.VectorSubcoreMesh(core_axis_name="c", subcore_axis_name="s")

    rows_t = [pltpu.VMEM((K, D), jnp.float32) for _ in range(NBUF)]
    srcb_t = [pltpu.VMEM((K,), jnp.int32) for _ in range(NBUF)]
    dstb_t = [pltpu.VMEM((K,), jnp.int32) for _ in range(NBUF)]
    valb_t = [pltpu.VMEM((K,), jnp.float32) for _ in range(NBUF)]
    sems_t = [pltpu.SemaphoreType.DMA for _ in range(3 * NBUF)]

    @functools.partial(
        pl.kernel,
        out_type=jax.ShapeDtypeStruct((NC, N_NODES, D), jnp.float32),
        mesh=mesh,
        scratch_types=(rows_t + srcb_t + dstb_t + valb_t
                       + [pltpu.VMEM((ZROWS, D), jnp.float32),
                          pltpu.VMEM_SHARED((N_NODES, D), jnp.float32)]
                       + sems_t),
    )
    def k(x_hbm, src_hbm, dst_hbm, vals_hbm, out_hbm, *refs):
        rows = refs[0:NBUF]
        srcb = refs[NBUF:2 * NBUF]
        dstb = refs[2 * NBUF:3 * NBUF]
        valb = refs[3 * NBUF:4 * NBUF]
        zero_v = refs[4 * NBUF]
        acc_sh = refs[4 * NBUF + 1]
        rsem = refs[4 * NBUF + 2:4 * NBUF + 2 + NBUF]
        ssem = refs[4 * NBUF + 2 + NBUF:4 * NBUF + 2 + 2 * NBUF]
        msem = refs[4 * NBUF + 2 + 2 * NBUF:4 * NBUF + 2 + 3 * NBUF]

        cid = lax.axis_index("c")
        sid = lax.axis_index("s")
        wid = sid * NC + cid
        base0 = wid * EPW

        def meta_fetch(c, q):
            sl = pl.ds(base0 + c * K, K)
            pltpu.async_copy(src_hbm.at[sl], srcb[q], msem[q])
            pltpu.async_copy(dst_hbm.at[sl], dstb[q], msem[q])
            pltpu.async_copy(vals_hbm.at[sl], valb[q], msem[q])

        def meta_wait(c, q):
            sl = pl.ds(base0 + c * K, K)
            pltpu.make_async_copy(src_hbm.at[sl], srcb[q], msem[q]).wait()
            pltpu.make_async_copy(dst_hbm.at[sl], dstb[q], msem[q]).wait()
            pltpu.make_async_copy(vals_hbm.at[sl], valb[q], msem[q]).wait()

        def gather(q):
            pltpu.async_copy(x_hbm.at[srcb[q]], rows[q], rsem[q])

        def gather_wait(q):
            pltpu.make_async_copy(x_hbm.at[srcb[q]], rows[q],
                                  rsem[q]).wait()

        def scatter(q):
            pltpu.async_copy(rows[q], acc_sh.at[dstb[q]], ssem[q], add=True)

        def scatter_wait(q):
            pltpu.make_async_copy(rows[q], acc_sh.at[dstb[q]],
                                  ssem[q]).wait()

        def scale(q):
            buf = rows[q]
            vbuf = valb[q]

            def body(g, _):
                vv = vbuf[pl.ds(g * L, L)]
                for i in range(L):
                    v = vv[i]
                    e = g * L + i
                    for j in range(D // L):
                        sl = pl.ds(j * L, L)
                        buf[e, sl] = buf[e, sl] * v
                return 0

            lax.fori_loop(0, K // L, body, 0)

        meta_fetch(0, 0)
        meta_fetch(1, 1)

        zvec = jnp.zeros((L,), jnp.float32)
        for j in range(D // L):
            for i in range(ZROWS):
                zero_v[i, pl.ds(j * L, L)] = zvec
        row0 = pl.multiple_of(sid * RPT, 8)

        def zcopy(t, _):
            off = pl.multiple_of(row0 + t * ZROWS, 8)
            pltpu.sync_copy(zero_v, acc_sh.at[pl.ds(off, ZROWS)])
            return 0

        lax.fori_loop(0, RPT // ZROWS, zcopy, 0)

        @pl.when(sid == NS - 1)
        def _zero_tail():
            pltpu.sync_copy(zero_v, acc_sh.at[pl.ds(RPT * NS, TAIL)])

        meta_wait(0, 0)
        gather(0)
        meta_wait(1, 1)
        gather(1)
        plsc.subcore_barrier()

        def step(s, _):
            c0 = s * NBUF
            for q in range(NBUF):
                c = c0 + q
                f = (q + 2) % NBUF

                @pl.when(c + 2 < CHUNKS)
                def _prefetch():
                    meta_fetch(c + 2, f)

                gather_wait(q)
                scale(q)

                @pl.when(c >= 1)
                def _drain_prev():
                    scatter_wait((q + NBUF - 1) % NBUF)

                scatter(q)

                @pl.when(c + 2 < CHUNKS)
                def _launch():
                    meta_wait(c + 2, f)
                    gather(f)

            return 0

        lax.fori_loop(0, STEPS, step, 0)

        gather_wait(0)
        scale(0)
        scatter_wait(3)
        scatter(0)
        scatter_wait(0)
        plsc.subcore_barrier()

        pltpu.sync_copy(acc_sh.at[pl.ds(row0, RPT)],
                        out_hbm.at[cid, pl.ds(row0, RPT)])

        @pl.when(sid == NS - 1)
        def _write_tail():
            pltpu.sync_copy(acc_sh.at[pl.ds(RPT * NS, TAIL)],
                            out_hbm.at[cid, pl.ds(RPT * NS, TAIL)])

    return k(x, src, dst, vals)


def _tc_combine(partials, W, b):
    R = 1000

    def body(p0_ref, p1_ref, w_ref, b_ref, o_ref):
        s = p0_ref[...] + p1_ref[...]
        y = jnp.dot(s, w_ref[...], preferred_element_type=jnp.float32)
        o_ref[...] = jnp.maximum(y + b_ref[...], 0.0)

    return pl.pallas_call(
        body,
        grid=(N_NODES // R,),
        in_specs=[
            pl.BlockSpec((R, D), lambda i: (i, 0)),
            pl.BlockSpec((R, D), lambda i: (i, 0)),
            pl.BlockSpec((D, D), lambda i: (0, 0)),
            pl.BlockSpec((1, D), lambda i: (0, 0)),
        ],
        out_specs=pl.BlockSpec((R, D), lambda i: (i, 0)),
        out_shape=jax.ShapeDtypeStruct((N_NODES, D), jnp.float32),
    )(partials[0], partials[1], W, b.reshape(1, D))


def kernel(x, edge_index, edge_vals, W, b):
    src = edge_index[0].astype(jnp.int32)
    dst = edge_index[1].astype(jnp.int32)
    partials = _sc_scatter(x, src, dst, edge_vals.astype(jnp.float32))
    return _tc_combine(partials, W, b)


# speedup vs baseline: 2.7284x; 1.0005x over previous
import functools

import jax
import jax.numpy as jnp
from jax import lax
from jax.experimental import pallas as pl
from jax.experimental.pallas import tpu as pltpu
from jax.experimental.pallas import tpu_sc as plsc

N_NODES = 10000
N_EDGES = 320000
D = 128
L = 16

NC = 2
NS = 16
NW = NC * NS
EPW = N_EDGES // NW
K = 80
CHUNKS = EPW // K
NBUF = 4
STEPS = (CHUNKS - 1) // NBUF
RPT = 624
TAIL = N_NODES - RPT * NS
ZROWS = 16


def _sc_scatter(x, src, dst, vals):
    mesh = plsc.VectorSubcoreMesh(core_axis_name="c", subcore_axis_name="s")

    rows_t = [pltpu.VMEM((K, D), jnp.float32) for _ in range(NBUF)]
    srcb_t = [pltpu.VMEM((K,), jnp.int32) for _ in range(NBUF)]
    dstb_t = [pltpu.VMEM((K,), jnp.int32) for _ in range(NBUF)]
    valb_t = [pltpu.VMEM((K,), jnp.float32) for _ in range(NBUF)]
    sems_t = [pltpu.SemaphoreType.DMA for _ in range(3 * NBUF)]

    @functools.partial(
        pl.kernel,
        out_type=jax.ShapeDtypeStruct((NC, N_NODES, D), jnp.float32),
        mesh=mesh,
        scratch_types=(rows_t + srcb_t + dstb_t + valb_t
                       + [pltpu.VMEM((ZROWS, D), jnp.float32),
                          pltpu.VMEM_SHARED((N_NODES, D), jnp.float32)]
                       + sems_t),
    )
    def k(x_hbm, src_hbm, dst_hbm, vals_hbm, out_hbm, *refs):
        rows = refs[0:NBUF]
        srcb = refs[NBUF:2 * NBUF]
        dstb = refs[2 * NBUF:3 * NBUF]
        valb = refs[3 * NBUF:4 * NBUF]
        zero_v = refs[4 * NBUF]
        acc_sh = refs[4 * NBUF + 1]
        rsem = refs[4 * NBUF + 2:4 * NBUF + 2 + NBUF]
        ssem = refs[4 * NBUF + 2 + NBUF:4 * NBUF + 2 + 2 * NBUF]
        msem = refs[4 * NBUF + 2 + 2 * NBUF:4 * NBUF + 2 + 3 * NBUF]

        cid = lax.axis_index("c")
        sid = lax.axis_index("s")
        wid = sid * NC + cid
        base0 = wid * EPW

        def meta_fetch(c, q):
            sl = pl.ds(base0 + c * K, K)
            pltpu.async_copy(src_hbm.at[sl], srcb[q], msem[q])
            pltpu.async_copy(dst_hbm.at[sl], dstb[q], msem[q])
            pltpu.async_copy(vals_hbm.at[sl], valb[q], msem[q])

        def meta_wait(c, q):
            sl = pl.ds(base0 + c * K, K)
            pltpu.make_async_copy(src_hbm.at[sl], srcb[q], msem[q]).wait()
            pltpu.make_async_copy(dst_hbm.at[sl], dstb[q], msem[q]).wait()
            pltpu.make_async_copy(vals_hbm.at[sl], valb[q], msem[q]).wait()

        def gather(q):
            pltpu.async_copy(x_hbm.at[srcb[q]], rows[q], rsem[q])

        def gather_wait(q):
            pltpu.make_async_copy(x_hbm.at[srcb[q]], rows[q],
                                  rsem[q]).wait()

        def scatter(q):
            pltpu.async_copy(rows[q], acc_sh.at[dstb[q]], ssem[q], add=True)

        def scatter_wait(q):
            pltpu.make_async_copy(rows[q], acc_sh.at[dstb[q]],
                                  ssem[q]).wait()

        def scale(q):
            buf = rows[q]
            vbuf = valb[q]

            def body(g, _):
                vv = vbuf[pl.ds(g * L, L)]
                for i in range(L):
                    v = vv[i]
                    e = g * L + i
                    for j in range(D // L):
                        sl = pl.ds(j * L, L)
                        buf[e, sl] = buf[e, sl] * v
                return 0

            lax.fori_loop(0, K // L, body, 0)

        meta_fetch(0, 0)
        meta_fetch(1, 1)
        meta_fetch(2, 2)

        zvec = jnp.zeros((L,), jnp.float32)
        for j in range(D // L):
            for i in range(ZROWS):
                zero_v[i, pl.ds(j * L, L)] = zvec
        row0 = pl.multiple_of(sid * RPT, 8)

        def zcopy(t, _):
            off = pl.multiple_of(row0 + t * ZROWS, 8)
            pltpu.sync_copy(zero_v, acc_sh.at[pl.ds(off, ZROWS)])
            return 0

        lax.fori_loop(0, RPT // ZROWS, zcopy, 0)

        @pl.when(sid == NS - 1)
        def _zero_tail():
            pltpu.sync_copy(zero_v, acc_sh.at[pl.ds(RPT * NS, TAIL)])

        meta_wait(0, 0)
        gather(0)
        meta_wait(1, 1)
        gather(1)
        plsc.subcore_barrier()

        def step(s, _):
            c0 = s * NBUF
            for q in range(NBUF):
                c = c0 + q
                f = (q + 2) % NBUF

                gather_wait(q)
                scale(q)

                @pl.when(c >= 1)
                def _drain_prev():
                    scatter_wait((q + NBUF - 1) % NBUF)

                scatter(q)

                @pl.when(c + 3 < CHUNKS)
                def _prefetch():
                    meta_fetch(c + 3, (q + 3) % NBUF)

                @pl.when(c + 2 < CHUNKS)
                def _launch():
                    meta_wait(c + 2, f)
                    gather(f)

            return 0

        lax.fori_loop(0, STEPS, step, 0)

        gather_wait(0)
        scale(0)
        scatter_wait(3)
        scatter(0)
        scatter_wait(0)
        plsc.subcore_barrier()

        pltpu.sync_copy(acc_sh.at[pl.ds(row0, RPT)],
                        out_hbm.at[cid, pl.ds(row0, RPT)])

        @pl.when(sid == NS - 1)
        def _write_tail():
            pltpu.sync_copy(acc_sh.at[pl.ds(RPT * NS, TAIL)],
                            out_hbm.at[cid, pl.ds(RPT * NS, TAIL)])

    return k(x, src, dst, vals)


def _tc_combine(partials, W, b):
    R = 1000

    def body(p0_ref, p1_ref, w_ref, b_ref, o_ref):
        s = p0_ref[...] + p1_ref[...]
        y = jnp.dot(s, w_ref[...], preferred_element_type=jnp.float32)
        o_ref[...] = jnp.maximum(y + b_ref[...], 0.0)

    return pl.pallas_call(
        body,
        grid=(N_NODES // R,),
        in_specs=[
            pl.BlockSpec((R, D), lambda i: (i, 0)),
            pl.BlockSpec((R, D), lambda i: (i, 0)),
            pl.BlockSpec((D, D), lambda i: (0, 0)),
            pl.BlockSpec((1, D), lambda i: (0, 0)),
        ],
        out_specs=pl.BlockSpec((R, D), lambda i: (i, 0)),
        out_shape=jax.ShapeDtypeStruct((N_NODES, D), jnp.float32),
    )(partials[0], partials[1], W, b.reshape(1, D))


def kernel(x, edge_index, edge_vals, W, b):
    src = edge_index[0].astype(jnp.int32)
    dst = edge_index[1].astype(jnp.int32)
    partials = _sc_scatter(x, src, dst, edge_vals.astype(jnp.float32))
    return _tc_combine(partials, W, b)
